# baseline (speedup 1.0000x reference)
import functools

import jax
import jax.numpy as jnp
from jax import lax
from jax.experimental import pallas as pl
from jax.experimental.pallas import tpu as pltpu
from jax.experimental.pallas import tpu_sc as plsc

EMBED_DIM = 64
BATCH = 1024
SEQ_LEN = 200
N_TOK = BATCH * SEQ_LEN
N_WORKERS = 32
PER_W = N_TOK // N_WORKERS
CHUNK = 800
N_CHUNK = PER_W // CHUNK


def _sc_gather(table_hbm, idx_hbm, out_hbm, idx_v, rows_v, sem):
    wid = lax.axis_index("s") * 2 + lax.axis_index("c")
    base = wid * PER_W

    def body(i, carry):
        off = base + i * CHUNK
        pltpu.sync_copy(idx_hbm.at[pl.ds(off, CHUNK)], idx_v)
        pltpu.async_copy(table_hbm.at[idx_v], rows_v, sem).wait()
        pltpu.sync_copy(rows_v, out_hbm.at[pl.ds(off, CHUNK)])
        return carry

    lax.fori_loop(0, N_CHUNK, body, 0)


_gather_call = functools.partial(
    pl.kernel,
    mesh=plsc.VectorSubcoreMesh(core_axis_name="c", subcore_axis_name="s"),
    out_type=jax.ShapeDtypeStruct((N_TOK, EMBED_DIM), jnp.float32),
    scratch_types=[
        pltpu.VMEM((CHUNK,), jnp.int32),
        pltpu.VMEM((CHUNK, EMBED_DIM), jnp.float32),
        pltpu.SemaphoreType.DMA,
    ],
    compiler_params=pltpu.CompilerParams(use_tc_tiling_on_sc=False),
)(_sc_gather)


def _mask_body(tokens_ref, pad_ref, seq_ref):
    pad_ref[...] = tokens_ref[...] == 0
    row = lax.broadcasted_iota(jnp.int32, (SEQ_LEN, SEQ_LEN), 0)
    col = lax.broadcasted_iota(jnp.int32, (SEQ_LEN, SEQ_LEN), 1)
    seq_ref[...] = col > row


_mask_call = pl.pallas_call(
    _mask_body,
    out_shape=(
        jax.ShapeDtypeStruct((BATCH, SEQ_LEN), jnp.bool_),
        jax.ShapeDtypeStruct((SEQ_LEN, SEQ_LEN), jnp.bool_),
    ),
)


@jax.jit
def kernel(tokens, table):
    tokens = tokens.astype(jnp.int32)
    idx = tokens.reshape(N_TOK)
    features = _gather_call(table, idx).reshape(BATCH, SEQ_LEN, EMBED_DIM)
    pad, seq = _mask_call(tokens)
    return (features, (pad[:, None, :], seq))


# speedup vs baseline: 1.0061x; 1.0061x over previous
import functools

import jax
import jax.numpy as jnp
from jax import lax
from jax.experimental import pallas as pl
from jax.experimental.pallas import tpu as pltpu
from jax.experimental.pallas import tpu_sc as plsc

EMBED_DIM = 64
BATCH = 1024
SEQ_LEN = 200
N_TOK = BATCH * SEQ_LEN
N_WORKERS = 32
PER_W = N_TOK // N_WORKERS
CHUNK = 800
N_CHUNK = PER_W // CHUNK


NBUF = 2


def _sc_gather(table_hbm, idx_hbm, out_hbm, idx_v,
               rows0, rows1, gsem0, gsem1, osem0, osem1):
    wid = lax.axis_index("s") * 2 + lax.axis_index("c")
    base = wid * PER_W

    rows = [rows0, rows1]
    gsems = [gsem0, gsem1]
    osems = [osem0, osem1]

    pltpu.sync_copy(idx_hbm.at[pl.ds(wid * N_CHUNK, N_CHUNK)], idx_v)

    gcp = [None] * N_CHUNK
    ocp = [None] * N_CHUNK
    for j in range(N_CHUNK + 1):
        if j < N_CHUNK:
            b = j % NBUF
            if j >= NBUF:
                ocp[j - NBUF].wait()
            gcp[j] = pltpu.async_copy(
                table_hbm.at[idx_v.at[j]], rows[b], gsems[b])
        if j >= 1:
            k = j - 1
            b = k % NBUF
            gcp[k].wait()
            ocp[k] = pltpu.async_copy(
                rows[b], out_hbm.at[pl.ds(base + k * CHUNK, CHUNK)], osems[b])
    for k in range(N_CHUNK - NBUF, N_CHUNK):
        ocp[k].wait()


_gather_call = functools.partial(
    pl.kernel,
    mesh=plsc.VectorSubcoreMesh(core_axis_name="c", subcore_axis_name="s"),
    out_type=jax.ShapeDtypeStruct((N_TOK, EMBED_DIM), jnp.float32),
    scratch_types=[
        pltpu.VMEM((N_CHUNK, CHUNK), jnp.int32),
        pltpu.VMEM((CHUNK, EMBED_DIM), jnp.float32),
        pltpu.VMEM((CHUNK, EMBED_DIM), jnp.float32),
        pltpu.SemaphoreType.DMA,
        pltpu.SemaphoreType.DMA,
        pltpu.SemaphoreType.DMA,
        pltpu.SemaphoreType.DMA,
    ],
    compiler_params=pltpu.CompilerParams(use_tc_tiling_on_sc=False),
)(_sc_gather)


def _mask_body(tokens_ref, pad_ref, seq_ref):
    pad_ref[...] = tokens_ref[...] == 0
    row = lax.broadcasted_iota(jnp.int32, (SEQ_LEN, SEQ_LEN), 0)
    col = lax.broadcasted_iota(jnp.int32, (SEQ_LEN, SEQ_LEN), 1)
    seq_ref[...] = col > row


_mask_call = pl.pallas_call(
    _mask_body,
    out_shape=(
        jax.ShapeDtypeStruct((BATCH, SEQ_LEN), jnp.bool_),
        jax.ShapeDtypeStruct((SEQ_LEN, SEQ_LEN), jnp.bool_),
    ),
)


@jax.jit
def kernel(tokens, table):
    tokens = tokens.astype(jnp.int32)
    idx = tokens.reshape(N_WORKERS * N_CHUNK, CHUNK)
    features = _gather_call(table, idx).reshape(BATCH, SEQ_LEN, EMBED_DIM)
    pad, seq = _mask_call(tokens)
    return (features, (pad[:, None, :], seq))
